# Initial kernel scaffold; baseline (speedup 1.0000x reference)
import jax
import jax.numpy as jnp
from jax.experimental import pallas as pl


def kernel(x, token_table, pos_table):
    raise NotImplementedError("write your pallas kernel here")


# speedup vs baseline: 7.6620x; 7.6620x over previous
import functools

import jax
import jax.numpy as jnp
from jax import lax
from jax.experimental import pallas as pl
from jax.experimental.pallas import tpu as pltpu
from jax.experimental.pallas import tpu_sc as plsc

D = 128
SEQ = 200
BATCH = 1024
L = 16

NC = 2
NS = 16
NW = NC * NS
B_PER_W = BATCH // NW

_CHUNKS = ((0, 128), (128, 72))


def _emb_body(x_hbm, tok_hbm, pos_hbm, out_hbm, pos_v, idx_v, rows_v, sem):
    wid = lax.axis_index("s") * NC + lax.axis_index("c")
    pltpu.sync_copy(pos_hbm, pos_v)

    def batch_body(i, carry):
        b = wid * B_PER_W + i
        pltpu.sync_copy(x_hbm.at[b], idx_v)
        copies = [
            pltpu.async_copy(
                tok_hbm.at[idx_v.at[pl.ds(off, n)]],
                rows_v.at[pl.ds(off, n)],
                sem,
            )
            for off, n in _CHUNKS
        ]
        for cp in copies:
            cp.wait()

        def row_body(r, c):
            for j in range(D // L):
                sl = pl.ds(j * L, L)
                rows_v[r, sl] = rows_v[r, sl] + pos_v[r, sl]
            return c

        lax.fori_loop(0, SEQ, row_body, 0)
        pltpu.sync_copy(rows_v, out_hbm.at[b])
        return carry

    lax.fori_loop(0, B_PER_W, batch_body, 0)


@functools.partial(
    pl.kernel,
    mesh=plsc.VectorSubcoreMesh(core_axis_name="c", subcore_axis_name="s"),
    out_type=jax.ShapeDtypeStruct((BATCH, SEQ, D), jnp.float32),
    scratch_types=[
        pltpu.VMEM((SEQ, D), jnp.float32),
        pltpu.VMEM((SEQ,), jnp.int32),
        pltpu.VMEM((SEQ, D), jnp.float32),
        pltpu.SemaphoreType.DMA,
    ],
)
def _emb_kernel(x_hbm, tok_hbm, pos_hbm, out_hbm, pos_v, idx_v, rows_v, sem):
    _emb_body(x_hbm, tok_hbm, pos_hbm, out_hbm, pos_v, idx_v, rows_v, sem)


def kernel(x, token_table, pos_table):
    return _emb_kernel(x, token_table, pos_table)


# speedup vs baseline: 14.2291x; 1.8571x over previous
import functools

import jax
import jax.numpy as jnp
from jax import lax
from jax.experimental import pallas as pl
from jax.experimental.pallas import tpu as pltpu
from jax.experimental.pallas import tpu_sc as plsc

D = 128
SEQ = 200
BATCH = 1024
L = 16

NC = 2
NS = 16
NW = NC * NS
B_PER_W = BATCH // NW

NBUF = 4

_CHUNKS = ((0, 128), (128, 72))


def _emb_body(x_hbm, tok_hbm, pos_hbm, out_hbm, pos_v, idxs, rows, sem_i,
              sem_g, sem_w):
    wid = lax.axis_index("s") * NC + lax.axis_index("c")
    base = wid * B_PER_W
    pltpu.sync_copy(pos_hbm, pos_v)

    def fire_idx(i):
        return pltpu.async_copy(x_hbm.at[base + i], idxs[i % NBUF], sem_i)

    def fire_gathers(i):
        buf = i % NBUF
        return [
            pltpu.async_copy(
                tok_hbm.at[idxs[buf].at[pl.ds(off, n)]],
                rows[buf].at[pl.ds(off, n)],
                sem_g,
            )
            for off, n in _CHUNKS
        ]

    def add_pos(i):
        buf = i % NBUF

        def row_body(r, c):
            for j in range(D // L):
                sl = pl.ds(j * L, L)
                rows[buf][r, sl] = rows[buf][r, sl] + pos_v[r, sl]
            return c

        lax.fori_loop(0, SEQ, row_body, 0)

    def fire_write(i):
        return pltpu.async_copy(rows[i % NBUF], out_hbm.at[base + i], sem_w)

    idx_cps = {i: fire_idx(i) for i in range(min(3, B_PER_W))}
    gather_cps = {}
    write_cps = {}
    for i in range(min(2, B_PER_W)):
        idx_cps.pop(i).wait()
        gather_cps[i] = fire_gathers(i)

    for i in range(B_PER_W):
        if i + 3 < B_PER_W:
            idx_cps[i + 3] = fire_idx(i + 3)
        if i + 2 < B_PER_W:
            if i - 2 >= 0:
                write_cps.pop(i - 2).wait()
            idx_cps.pop(i + 2).wait()
            gather_cps[i + 2] = fire_gathers(i + 2)
        for cp in gather_cps.pop(i):
            cp.wait()
        add_pos(i)
        write_cps[i] = fire_write(i)

    for i in sorted(write_cps):
        write_cps.pop(i).wait()


@functools.partial(
    pl.kernel,
    mesh=plsc.VectorSubcoreMesh(core_axis_name="c", subcore_axis_name="s"),
    out_type=jax.ShapeDtypeStruct((BATCH, SEQ, D), jnp.float32),
    scratch_types=[
        pltpu.VMEM((SEQ, D), jnp.float32),
        [pltpu.VMEM((SEQ,), jnp.int32) for _ in range(NBUF)],
        [pltpu.VMEM((SEQ, D), jnp.float32) for _ in range(NBUF)],
        pltpu.SemaphoreType.DMA,
        pltpu.SemaphoreType.DMA,
        pltpu.SemaphoreType.DMA,
    ],
)
def _emb_kernel(x_hbm, tok_hbm, pos_hbm, out_hbm, pos_v, idxs, rows, sem_i,
                sem_g, sem_w):
    _emb_body(x_hbm, tok_hbm, pos_hbm, out_hbm, pos_v, idxs, rows, sem_i,
              sem_g, sem_w)


def kernel(x, token_table, pos_table):
    return _emb_kernel(x, token_table, pos_table)


# speedup vs baseline: 14.3702x; 1.0099x over previous
import functools

import jax
import jax.numpy as jnp
from jax import lax
from jax.experimental import pallas as pl
from jax.experimental.pallas import tpu as pltpu
from jax.experimental.pallas import tpu_sc as plsc

D = 128
SEQ = 200
BATCH = 1024
L = 16

NC = 2
NS = 16
NW = NC * NS
B_PER_W = BATCH // NW

NBUF = 4

_CHUNKS = ((0, 128), (128, 72))


def _emb_body(x_hbm, tok_hbm, pos_hbm, out_hbm, pos_v, idxs, rows, sem_i,
              sem_g, sem_w):
    wid = lax.axis_index("s") * NC + lax.axis_index("c")
    base = wid * B_PER_W
    pos_cp = pltpu.async_copy(pos_hbm, pos_v, sem_w)

    def fire_idx(i):
        return pltpu.async_copy(x_hbm.at[base + i], idxs[i % NBUF], sem_i)

    def fire_gathers(i):
        buf = i % NBUF
        return [
            pltpu.async_copy(
                tok_hbm.at[idxs[buf].at[pl.ds(off, n)]],
                rows[buf].at[pl.ds(off, n)],
                sem_g,
            )
            for off, n in _CHUNKS
        ]

    def add_pos(i):
        buf = i % NBUF

        def row_body(r, c):
            for j in range(D // L):
                sl = pl.ds(j * L, L)
                rows[buf][r, sl] = rows[buf][r, sl] + pos_v[r, sl]
            return c

        lax.fori_loop(0, SEQ, row_body, 0)

    def fire_write(i):
        return pltpu.async_copy(rows[i % NBUF], out_hbm.at[base + i], sem_w)

    idx_cps = {i: fire_idx(i) for i in range(min(3, B_PER_W))}
    gather_cps = {}
    write_cps = {}
    for i in range(min(2, B_PER_W)):
        idx_cps.pop(i).wait()
        gather_cps[i] = fire_gathers(i)
    pos_cp.wait()

    for i in range(B_PER_W):
        if i + 3 < B_PER_W:
            idx_cps[i + 3] = fire_idx(i + 3)
        if i + 2 < B_PER_W:
            if i - 2 >= 0:
                write_cps.pop(i - 2).wait()
            idx_cps.pop(i + 2).wait()
            gather_cps[i + 2] = fire_gathers(i + 2)
        for cp in gather_cps.pop(i):
            cp.wait()
        add_pos(i)
        write_cps[i] = fire_write(i)

    for i in sorted(write_cps):
        write_cps.pop(i).wait()


@functools.partial(
    pl.kernel,
    mesh=plsc.VectorSubcoreMesh(core_axis_name="c", subcore_axis_name="s"),
    out_type=jax.ShapeDtypeStruct((BATCH, SEQ, D), jnp.float32),
    scratch_types=[
        pltpu.VMEM((SEQ, D), jnp.float32),
        [pltpu.VMEM((SEQ,), jnp.int32) for _ in range(NBUF)],
        [pltpu.VMEM((SEQ, D), jnp.float32) for _ in range(NBUF)],
        pltpu.SemaphoreType.DMA,
        pltpu.SemaphoreType.DMA,
        pltpu.SemaphoreType.DMA,
    ],
)
def _emb_kernel(x_hbm, tok_hbm, pos_hbm, out_hbm, pos_v, idxs, rows, sem_i,
                sem_g, sem_w):
    _emb_body(x_hbm, tok_hbm, pos_hbm, out_hbm, pos_v, idxs, rows, sem_i,
              sem_g, sem_w)


def kernel(x, token_table, pos_table):
    return _emb_kernel(x, token_table, pos_table)
